# baseline (speedup 1.0000x reference)
import functools

import jax
import jax.numpy as jnp
from jax import lax
from jax.experimental import pallas as pl
from jax.experimental.pallas import tpu as pltpu
from jax.experimental.pallas import tpu_sc as plsc

N_PAD = 10240
N_TILES = 16
N_CORES = 2
N_WORKERS = N_CORES * N_TILES
ROWS_PER_TILE = N_PAD // N_TILES
E_CHUNK = 80
N_CHUNKS = 126
E_PAD = N_WORKERS * N_CHUNKS * E_CHUNK
PAD_NODE = 10200
F_HALF = 64
F2 = 8

def _worker_id():
    return lax.axis_index("s") * N_CORES + lax.axis_index("c")


def _mesh():
    return plsc.VectorSubcoreMesh(
        core_axis_name="c", subcore_axis_name="s",
        num_cores=N_CORES, num_subcores=N_TILES)


def _sc_degree_body(ei, degp, acc, idx_dst, ones_v, zeros_v):
    c = lax.axis_index("c")
    s = lax.axis_index("s")
    w = _worker_id()
    for i in range(E_CHUNK // 16):
        ones_v[pl.ds(i * 16, 16)] = jnp.ones((16,), jnp.float32)
    for i in range(ROWS_PER_TILE // 16):
        zeros_v[pl.ds(i * 16, 16)] = jnp.zeros((16,), jnp.float32)
    pltpu.sync_copy(zeros_v, acc.at[pl.ds(s * ROWS_PER_TILE, ROWS_PER_TILE)])
    pltpu.sync_copy(ei.at[1, w], idx_dst)
    plsc.subcore_barrier()

    def body(i, _):
        pltpu.sync_copy(ones_v, acc.at[idx_dst.at[i]], add=True)
        return ()

    lax.fori_loop(0, N_CHUNKS, body, ())
    plsc.subcore_barrier()
    pltpu.sync_copy(acc.at[pl.ds(s * ROWS_PER_TILE, ROWS_PER_TILE)],
                    degp.at[c, pl.ds(s * ROWS_PER_TILE, ROWS_PER_TILE)])


def _pipelined_agg(table, acc, idx_src, idx_dst, msg0, msg1,
                   semg0, semg1, sems0, sems1, n_chunks):
    assert n_chunks % 2 == 0

    def gather(i, buf, sem):
        pltpu.async_copy(table.at[idx_src.at[i]], buf, sem)

    def wait_gather(i, buf, sem):
        pltpu.make_async_copy(table.at[idx_src.at[i]], buf, sem).wait()

    def scatter(i, buf, sem):
        pltpu.async_copy(buf, acc.at[idx_dst.at[i]], sem, add=True)

    def wait_scatter(i, buf, sem):
        pltpu.make_async_copy(buf, acc.at[idx_dst.at[i]], sem).wait()

    gather(0, msg0, semg0)
    gather(1, msg1, semg1)

    def body(k, _):
        i0, i1, i2, i3 = 2 * k, 2 * k + 1, 2 * k + 2, 2 * k + 3
        wait_gather(i0, msg0, semg0)
        scatter(i0, msg0, sems0)
        wait_gather(i1, msg1, semg1)
        scatter(i1, msg1, sems1)

        @pl.when(i2 < n_chunks)
        def _():
            wait_scatter(i0, msg0, sems0)
            gather(i2, msg0, semg0)

        @pl.when(i3 < n_chunks)
        def _():
            wait_scatter(i1, msg1, sems1)
            gather(i3, msg1, semg1)

        return ()

    lax.fori_loop(0, n_chunks // 2, body, ())
    wait_scatter(n_chunks - 2, msg0, sems0)
    wait_scatter(n_chunks - 1, msg1, sems1)


def _sc_agg1_body(p1, ei1, agg, acc, idx_src, idx_dst, msg0, msg1,
                  semg0, semg1, sems0, sems1):
    c = lax.axis_index("c")
    s = lax.axis_index("s")
    rows = pl.ds(s * ROWS_PER_TILE, ROWS_PER_TILE)
    pltpu.sync_copy(p1.at[c, rows], acc.at[rows])
    pltpu.sync_copy(ei1.at[0, s], idx_src)
    pltpu.sync_copy(ei1.at[1, s], idx_dst)
    plsc.subcore_barrier()
    _pipelined_agg(p1.at[c], acc, idx_src, idx_dst, msg0, msg1,
                   semg0, semg1, sems0, sems1, 2 * N_CHUNKS)
    plsc.subcore_barrier()
    pltpu.sync_copy(acc.at[rows], agg.at[c, rows])


def _sc_agg2_body(p2, ei, agg, acc, idx_src, idx_dst, msg0, msg1,
                  semg0, semg1, sems0, sems1):
    c = lax.axis_index("c")
    s = lax.axis_index("s")
    w = _worker_id()
    rows = pl.ds(s * ROWS_PER_TILE, ROWS_PER_TILE)
    pltpu.sync_copy(p2.at[rows], acc.at[rows])
    pltpu.sync_copy(ei.at[0, w], idx_src)
    pltpu.sync_copy(ei.at[1, w], idx_dst)
    plsc.subcore_barrier()
    _pipelined_agg(p2, acc, idx_src, idx_dst, msg0, msg1,
                   semg0, semg1, sems0, sems1, N_CHUNKS)
    plsc.subcore_barrier()
    pltpu.sync_copy(acc.at[rows], agg.at[c, rows])


@functools.cache
def _sc_kernels():
    sc_degree = pl.kernel(
        _sc_degree_body,
        out_type=jax.ShapeDtypeStruct((N_CORES, N_PAD), jnp.float32),
        mesh=_mesh(),
        compiler_params=pltpu.CompilerParams(use_tc_tiling_on_sc=False),
        scratch_types=[
            pltpu.VMEM_SHARED((N_PAD,), jnp.float32),
            pltpu.VMEM((N_CHUNKS, E_CHUNK), jnp.int32),
            pltpu.VMEM((E_CHUNK,), jnp.float32),
            pltpu.VMEM((ROWS_PER_TILE,), jnp.float32),
        ],
    )
    sc_agg1 = pl.kernel(
        _sc_agg1_body,
        out_type=jax.ShapeDtypeStruct((N_CORES, N_PAD, F_HALF), jnp.float32),
        mesh=_mesh(),
        compiler_params=pltpu.CompilerParams(use_tc_tiling_on_sc=False),
        scratch_types=[
            pltpu.VMEM_SHARED((N_PAD, F_HALF), jnp.float32),
            pltpu.VMEM((2 * N_CHUNKS, E_CHUNK), jnp.int32),
            pltpu.VMEM((2 * N_CHUNKS, E_CHUNK), jnp.int32),
            pltpu.VMEM((E_CHUNK, F_HALF), jnp.float32),
            pltpu.VMEM((E_CHUNK, F_HALF), jnp.float32),
            pltpu.SemaphoreType.DMA,
            pltpu.SemaphoreType.DMA,
            pltpu.SemaphoreType.DMA,
            pltpu.SemaphoreType.DMA,
        ],
    )
    sc_agg2 = pl.kernel(
        _sc_agg2_body,
        out_type=jax.ShapeDtypeStruct((N_CORES, N_PAD, F2), jnp.float32),
        mesh=_mesh(),
        compiler_params=pltpu.CompilerParams(use_tc_tiling_on_sc=False),
        scratch_types=[
            pltpu.VMEM_SHARED((N_PAD, F2), jnp.float32),
            pltpu.VMEM((N_CHUNKS, E_CHUNK), jnp.int32),
            pltpu.VMEM((N_CHUNKS, E_CHUNK), jnp.int32),
            pltpu.VMEM((E_CHUNK, F2), jnp.float32),
            pltpu.VMEM((E_CHUNK, F2), jnp.float32),
            pltpu.SemaphoreType.DMA,
            pltpu.SemaphoreType.DMA,
            pltpu.SemaphoreType.DMA,
            pltpu.SemaphoreType.DMA,
        ],
    )
    return sc_degree, sc_agg1, sc_agg2


def _tc1_body(x_ref, w_ref, degp_ref, p1_ref, dinv_ref):
    deg = degp_ref[:, 0:1] + degp_ref[:, 1:2] + 1.0
    dinv = lax.rsqrt(deg)
    dinv_ref[...] = dinv
    h = jnp.dot(x_ref[...], w_ref[0], preferred_element_type=jnp.float32)
    p1_ref[...] = (h * dinv)[None]


def _tc2_body(agg_ref, dinv_ref, b1_ref, w2_ref, p2_ref):
    dinv = dinv_ref[...]
    h = jnp.concatenate([agg_ref[0], agg_ref[1]], axis=1)
    r = jax.nn.relu(h * dinv + b1_ref[...])
    p2_ref[...] = jnp.dot(r, w2_ref[...],
                          preferred_element_type=jnp.float32) * dinv


def _tc3_body(agg_ref, p2_ref, dinv_ref, b2_ref, out_ref):
    tot = agg_ref[0] + agg_ref[1] - p2_ref[...]
    out_ref[...] = tot * dinv_ref[...] + b2_ref[...]


def kernel(x, edge_index, W1, b1, W2, b2):
    n, in_ch = x.shape
    ei32 = edge_index.astype(jnp.int32)
    pad_ids = 10000 + jnp.arange(E_PAD - ei32.shape[1], dtype=jnp.int32) % 240
    ei32 = jnp.concatenate(
        [ei32, jnp.stack([pad_ids, pad_ids])], axis=1)
    ei = ei32.reshape(2, N_WORKERS, N_CHUNKS, E_CHUNK)
    ei1 = ei32.reshape(2, N_TILES, 2 * N_CHUNKS, E_CHUNK)
    x_pad = jnp.pad(x, ((0, N_PAD - n), (0, 0)))
    w2_pad = jnp.pad(W2, ((0, 0), (0, F2 - W2.shape[1])))
    b2_pad = jnp.pad(b2, (0, F2 - b2.shape[0])).reshape(1, F2)
    b1_row = b1.reshape(1, -1)
    w1_split = W1.reshape(in_ch, N_CORES, F_HALF).transpose(1, 0, 2)
    _sc_degree, _sc_agg1, _sc_agg2 = _sc_kernels()

    degp = _sc_degree(ei)
    degp2 = degp.T

    p1, dinv = pl.pallas_call(
        _tc1_body,
        grid=(N_CORES,),
        in_specs=[
            pl.BlockSpec((N_PAD, in_ch), lambda c: (0, 0)),
            pl.BlockSpec((1, in_ch, F_HALF), lambda c: (c, 0, 0)),
            pl.BlockSpec((N_PAD, 2), lambda c: (0, 0)),
        ],
        out_specs=[
            pl.BlockSpec((1, N_PAD, F_HALF), lambda c: (c, 0, 0)),
            pl.BlockSpec((N_PAD, 1), lambda c: (0, 0)),
        ],
        out_shape=[
            jax.ShapeDtypeStruct((N_CORES, N_PAD, F_HALF), jnp.float32),
            jax.ShapeDtypeStruct((N_PAD, 1), jnp.float32),
        ],
    )(x_pad, w1_split, degp2)

    agg1 = _sc_agg1(p1, ei1)

    p2 = pl.pallas_call(
        _tc2_body,
        out_shape=jax.ShapeDtypeStruct((N_PAD, F2), jnp.float32),
    )(agg1, dinv, b1_row, w2_pad)

    agg2 = _sc_agg2(p2, ei)

    out = pl.pallas_call(
        _tc3_body,
        out_shape=jax.ShapeDtypeStruct((N_PAD, F2), jnp.float32),
    )(agg2, p2, dinv, b2_pad)

    return out[:n, :3]


# speedup vs baseline: 1.2276x; 1.2276x over previous
import functools

import jax
import jax.numpy as jnp
from jax import lax
from jax.experimental import pallas as pl
from jax.experimental.pallas import tpu as pltpu
from jax.experimental.pallas import tpu_sc as plsc

N_PAD = 10240
N_TILES = 16
N_CORES = 2
N_WORKERS = N_CORES * N_TILES
ROWS_PER_TILE = N_PAD // N_TILES
E_CHUNK = 112
N_CHUNKS = 90
E_PAD = N_WORKERS * N_CHUNKS * E_CHUNK
PAD_NODE = 10200
F_HALF = 64
F2 = 8

def _worker_id():
    return lax.axis_index("s") * N_CORES + lax.axis_index("c")


def _mesh():
    return plsc.VectorSubcoreMesh(
        core_axis_name="c", subcore_axis_name="s",
        num_cores=N_CORES, num_subcores=N_TILES)


def _sc_degree_body(ei, degp, acc, idx_dst, ones_v, zeros_v):
    c = lax.axis_index("c")
    s = lax.axis_index("s")
    w = _worker_id()
    for i in range(E_CHUNK // 16):
        ones_v[pl.ds(i * 16, 16)] = jnp.ones((16,), jnp.float32)
    for i in range(ROWS_PER_TILE // 16):
        zeros_v[pl.ds(i * 16, 16)] = jnp.zeros((16,), jnp.float32)
    pltpu.sync_copy(zeros_v, acc.at[pl.ds(s * ROWS_PER_TILE, ROWS_PER_TILE)])
    pltpu.sync_copy(ei.at[1, w], idx_dst)
    plsc.subcore_barrier()

    def body(i, _):
        pltpu.sync_copy(ones_v, acc.at[idx_dst.at[i]], add=True)
        return ()

    lax.fori_loop(0, N_CHUNKS, body, ())
    plsc.subcore_barrier()
    pltpu.sync_copy(acc.at[pl.ds(s * ROWS_PER_TILE, ROWS_PER_TILE)],
                    degp.at[c, pl.ds(s * ROWS_PER_TILE, ROWS_PER_TILE)])


def _pipelined_agg(table, acc, idx_src, idx_dst, msg0, msg1,
                   semg0, semg1, sems0, sems1, n_chunks):
    assert n_chunks % 2 == 0

    def gather(i, buf, sem):
        pltpu.async_copy(table.at[idx_src.at[i]], buf, sem)

    def wait_gather(i, buf, sem):
        pltpu.make_async_copy(table.at[idx_src.at[i]], buf, sem).wait()

    def scatter(i, buf, sem):
        pltpu.async_copy(buf, acc.at[idx_dst.at[i]], sem, add=True)

    def wait_scatter(i, buf, sem):
        pltpu.make_async_copy(buf, acc.at[idx_dst.at[i]], sem).wait()

    def sync_scatter(i, buf):
        pltpu.sync_copy(buf, acc.at[idx_dst.at[i]], add=True)

    gather(0, msg0, semg0)

    def body(k, _):
        i0, i1, i2 = 2 * k, 2 * k + 1, 2 * k + 2
        gather(i1, msg1, semg1)
        wait_gather(i0, msg0, semg0)
        sync_scatter(i0, msg0)

        @pl.when(i2 < n_chunks)
        def _():
            gather(i2, msg0, semg0)

        wait_gather(i1, msg1, semg1)
        sync_scatter(i1, msg1)
        return ()

    lax.fori_loop(0, n_chunks // 2, body, ())


def _sc_agg1_body(p1, ei1, agg, acc, idx_src, idx_dst, msg0, msg1,
                  semg0, semg1, sems0, sems1):
    c = lax.axis_index("c")
    s = lax.axis_index("s")
    rows = pl.ds(s * ROWS_PER_TILE, ROWS_PER_TILE)
    pltpu.sync_copy(p1.at[c, rows], acc.at[rows])
    pltpu.sync_copy(ei1.at[0, s], idx_src)
    pltpu.sync_copy(ei1.at[1, s], idx_dst)
    plsc.subcore_barrier()
    _pipelined_agg(p1.at[c], acc, idx_src, idx_dst, msg0, msg1,
                   semg0, semg1, sems0, sems1, 2 * N_CHUNKS)
    plsc.subcore_barrier()
    pltpu.sync_copy(acc.at[rows], agg.at[c, rows])


def _sc_agg2_body(p2, ei, agg, acc, idx_src, idx_dst, msg0, msg1,
                  semg0, semg1, sems0, sems1):
    c = lax.axis_index("c")
    s = lax.axis_index("s")
    w = _worker_id()
    rows = pl.ds(s * ROWS_PER_TILE, ROWS_PER_TILE)
    pltpu.sync_copy(p2.at[rows], acc.at[rows])
    pltpu.sync_copy(ei.at[0, w], idx_src)
    pltpu.sync_copy(ei.at[1, w], idx_dst)
    plsc.subcore_barrier()
    _pipelined_agg(p2, acc, idx_src, idx_dst, msg0, msg1,
                   semg0, semg1, sems0, sems1, N_CHUNKS)
    plsc.subcore_barrier()
    pltpu.sync_copy(acc.at[rows], agg.at[c, rows])


@functools.cache
def _sc_kernels():
    sc_degree = pl.kernel(
        _sc_degree_body,
        out_type=jax.ShapeDtypeStruct((N_CORES, N_PAD), jnp.float32),
        mesh=_mesh(),
        compiler_params=pltpu.CompilerParams(use_tc_tiling_on_sc=False),
        scratch_types=[
            pltpu.VMEM_SHARED((N_PAD,), jnp.float32),
            pltpu.VMEM((N_CHUNKS, E_CHUNK), jnp.int32),
            pltpu.VMEM((E_CHUNK,), jnp.float32),
            pltpu.VMEM((ROWS_PER_TILE,), jnp.float32),
        ],
    )
    sc_agg1 = pl.kernel(
        _sc_agg1_body,
        out_type=jax.ShapeDtypeStruct((N_CORES, N_PAD, F_HALF), jnp.float32),
        mesh=_mesh(),
        compiler_params=pltpu.CompilerParams(use_tc_tiling_on_sc=False),
        scratch_types=[
            pltpu.VMEM_SHARED((N_PAD, F_HALF), jnp.float32),
            pltpu.VMEM((2 * N_CHUNKS, E_CHUNK), jnp.int32),
            pltpu.VMEM((2 * N_CHUNKS, E_CHUNK), jnp.int32),
            pltpu.VMEM((E_CHUNK, F_HALF), jnp.float32),
            pltpu.VMEM((E_CHUNK, F_HALF), jnp.float32),
            pltpu.SemaphoreType.DMA,
            pltpu.SemaphoreType.DMA,
            pltpu.SemaphoreType.DMA,
            pltpu.SemaphoreType.DMA,
        ],
    )
    sc_agg2 = pl.kernel(
        _sc_agg2_body,
        out_type=jax.ShapeDtypeStruct((N_CORES, N_PAD, F2), jnp.float32),
        mesh=_mesh(),
        compiler_params=pltpu.CompilerParams(use_tc_tiling_on_sc=False),
        scratch_types=[
            pltpu.VMEM_SHARED((N_PAD, F2), jnp.float32),
            pltpu.VMEM((N_CHUNKS, E_CHUNK), jnp.int32),
            pltpu.VMEM((N_CHUNKS, E_CHUNK), jnp.int32),
            pltpu.VMEM((E_CHUNK, F2), jnp.float32),
            pltpu.VMEM((E_CHUNK, F2), jnp.float32),
            pltpu.SemaphoreType.DMA,
            pltpu.SemaphoreType.DMA,
            pltpu.SemaphoreType.DMA,
            pltpu.SemaphoreType.DMA,
        ],
    )
    return sc_degree, sc_agg1, sc_agg2


def _tc1_body(x_ref, w_ref, degp_ref, p1_ref, dinv_ref):
    deg = degp_ref[:, 0:1] + degp_ref[:, 1:2] + 1.0
    dinv = lax.rsqrt(deg)
    dinv_ref[...] = dinv
    h = jnp.dot(x_ref[...], w_ref[0], preferred_element_type=jnp.float32)
    p1_ref[...] = (h * dinv)[None]


def _tc2_body(agg_ref, dinv_ref, b1_ref, w2_ref, p2_ref):
    dinv = dinv_ref[...]
    h = jnp.concatenate([agg_ref[0], agg_ref[1]], axis=1)
    r = jax.nn.relu(h * dinv + b1_ref[...])
    p2_ref[...] = jnp.dot(r, w2_ref[...],
                          preferred_element_type=jnp.float32) * dinv


def _tc3_body(agg_ref, p2_ref, dinv_ref, b2_ref, out_ref):
    tot = agg_ref[0] + agg_ref[1] - p2_ref[...]
    out_ref[...] = tot * dinv_ref[...] + b2_ref[...]


def kernel(x, edge_index, W1, b1, W2, b2):
    n, in_ch = x.shape
    ei32 = edge_index.astype(jnp.int32)
    pad_ids = 10000 + jnp.arange(E_PAD - ei32.shape[1], dtype=jnp.int32) % 240
    ei32 = jnp.concatenate(
        [ei32, jnp.stack([pad_ids, pad_ids])], axis=1)
    ei = ei32.reshape(2, N_WORKERS, N_CHUNKS, E_CHUNK)
    ei1 = ei32.reshape(2, N_TILES, 2 * N_CHUNKS, E_CHUNK)
    x_pad = jnp.pad(x, ((0, N_PAD - n), (0, 0)))
    w2_pad = jnp.pad(W2, ((0, 0), (0, F2 - W2.shape[1])))
    b2_pad = jnp.pad(b2, (0, F2 - b2.shape[0])).reshape(1, F2)
    b1_row = b1.reshape(1, -1)
    w1_split = W1.reshape(in_ch, N_CORES, F_HALF).transpose(1, 0, 2)
    _sc_degree, _sc_agg1, _sc_agg2 = _sc_kernels()

    degp = _sc_degree(ei)
    degp2 = degp.T

    p1, dinv = pl.pallas_call(
        _tc1_body,
        grid=(N_CORES,),
        in_specs=[
            pl.BlockSpec((N_PAD, in_ch), lambda c: (0, 0)),
            pl.BlockSpec((1, in_ch, F_HALF), lambda c: (c, 0, 0)),
            pl.BlockSpec((N_PAD, 2), lambda c: (0, 0)),
        ],
        out_specs=[
            pl.BlockSpec((1, N_PAD, F_HALF), lambda c: (c, 0, 0)),
            pl.BlockSpec((N_PAD, 1), lambda c: (0, 0)),
        ],
        out_shape=[
            jax.ShapeDtypeStruct((N_CORES, N_PAD, F_HALF), jnp.float32),
            jax.ShapeDtypeStruct((N_PAD, 1), jnp.float32),
        ],
    )(x_pad, w1_split, degp2)

    agg1 = _sc_agg1(p1, ei1)

    p2 = pl.pallas_call(
        _tc2_body,
        out_shape=jax.ShapeDtypeStruct((N_PAD, F2), jnp.float32),
    )(agg1, dinv, b1_row, w2_pad)

    agg2 = _sc_agg2(p2, ei)

    out = pl.pallas_call(
        _tc3_body,
        out_shape=jax.ShapeDtypeStruct((N_PAD, F2), jnp.float32),
    )(agg2, p2, dinv, b2_pad)

    return out[:n, :3]


# speedup vs baseline: 1.2548x; 1.0222x over previous
import functools

import jax
import jax.numpy as jnp
from jax import lax
from jax.experimental import pallas as pl
from jax.experimental.pallas import tpu as pltpu
from jax.experimental.pallas import tpu_sc as plsc

N_PAD = 10240
N_TILES = 16
N_CORES = 2
N_WORKERS = N_CORES * N_TILES
ROWS_PER_TILE = N_PAD // N_TILES
E_CHUNK = 120
N_CHUNKS = 84
E_PAD = N_WORKERS * N_CHUNKS * E_CHUNK
PAD_NODE = 10200
F_HALF = 64
F2 = 8

def _worker_id():
    return lax.axis_index("s") * N_CORES + lax.axis_index("c")


def _mesh():
    return plsc.VectorSubcoreMesh(
        core_axis_name="c", subcore_axis_name="s",
        num_cores=N_CORES, num_subcores=N_TILES)


def _sc_degree_body(ei, degp, acc, idx_dst, ones_v, zeros_v):
    c = lax.axis_index("c")
    s = lax.axis_index("s")
    w = _worker_id()
    for i in range(E_CHUNK // 16):
        ones_v[pl.ds(i * 16, 16)] = jnp.ones((16,), jnp.float32)
    for i in range(ROWS_PER_TILE // 16):
        zeros_v[pl.ds(i * 16, 16)] = jnp.zeros((16,), jnp.float32)
    pltpu.sync_copy(zeros_v, acc.at[pl.ds(s * ROWS_PER_TILE, ROWS_PER_TILE)])
    pltpu.sync_copy(ei.at[1, w], idx_dst)
    plsc.subcore_barrier()

    def body(i, _):
        pltpu.sync_copy(ones_v, acc.at[idx_dst.at[i]], add=True)
        return ()

    lax.fori_loop(0, N_CHUNKS, body, ())
    plsc.subcore_barrier()
    pltpu.sync_copy(acc.at[pl.ds(s * ROWS_PER_TILE, ROWS_PER_TILE)],
                    degp.at[c, pl.ds(s * ROWS_PER_TILE, ROWS_PER_TILE)])


def _pipelined_agg(table, acc, idx_src, idx_dst, msg0, msg1,
                   semg0, semg1, sems0, sems1, n_chunks):
    assert n_chunks % 2 == 0

    def gather(i, buf, sem):
        pltpu.async_copy(table.at[idx_src.at[i]], buf, sem)

    def wait_gather(i, buf, sem):
        pltpu.make_async_copy(table.at[idx_src.at[i]], buf, sem).wait()

    def scatter(i, buf, sem):
        pltpu.async_copy(buf, acc.at[idx_dst.at[i]], sem, add=True)

    def wait_scatter(i, buf, sem):
        pltpu.make_async_copy(buf, acc.at[idx_dst.at[i]], sem).wait()

    def sync_scatter(i, buf):
        pltpu.sync_copy(buf, acc.at[idx_dst.at[i]], add=True)

    gather(0, msg0, semg0)

    def body(k, _):
        i0, i1, i2 = 2 * k, 2 * k + 1, 2 * k + 2
        gather(i1, msg1, semg1)
        wait_gather(i0, msg0, semg0)
        sync_scatter(i0, msg0)

        @pl.when(i2 < n_chunks)
        def _():
            gather(i2, msg0, semg0)

        wait_gather(i1, msg1, semg1)
        sync_scatter(i1, msg1)
        return ()

    lax.fori_loop(0, n_chunks // 2, body, ())


def _sc_agg1_body(p1, ei1, agg, acc, idx_src, idx_dst, msg0, msg1,
                  semg0, semg1, sems0, sems1):
    c = lax.axis_index("c")
    s = lax.axis_index("s")
    rows = pl.ds(s * ROWS_PER_TILE, ROWS_PER_TILE)
    pltpu.sync_copy(p1.at[c, rows], acc.at[rows])
    pltpu.sync_copy(ei1.at[0, s], idx_src)
    pltpu.sync_copy(ei1.at[1, s], idx_dst)
    plsc.subcore_barrier()
    _pipelined_agg(p1.at[c], acc, idx_src, idx_dst, msg0, msg1,
                   semg0, semg1, sems0, sems1, 2 * N_CHUNKS)
    plsc.subcore_barrier()
    pltpu.sync_copy(acc.at[rows], agg.at[c, rows])


def _sc_agg2_body(p2, ei, agg, acc, idx_src, idx_dst, msg0, msg1,
                  semg0, semg1, sems0, sems1):
    c = lax.axis_index("c")
    s = lax.axis_index("s")
    w = _worker_id()
    rows = pl.ds(s * ROWS_PER_TILE, ROWS_PER_TILE)
    pltpu.sync_copy(p2.at[rows], acc.at[rows])
    pltpu.sync_copy(ei.at[0, w], idx_src)
    pltpu.sync_copy(ei.at[1, w], idx_dst)
    plsc.subcore_barrier()
    _pipelined_agg(p2, acc, idx_src, idx_dst, msg0, msg1,
                   semg0, semg1, sems0, sems1, N_CHUNKS)
    plsc.subcore_barrier()
    pltpu.sync_copy(acc.at[rows], agg.at[c, rows])


@functools.cache
def _sc_kernels():
    sc_degree = pl.kernel(
        _sc_degree_body,
        out_type=jax.ShapeDtypeStruct((N_CORES, N_PAD), jnp.float32),
        mesh=_mesh(),
        compiler_params=pltpu.CompilerParams(use_tc_tiling_on_sc=False),
        scratch_types=[
            pltpu.VMEM_SHARED((N_PAD,), jnp.float32),
            pltpu.VMEM((N_CHUNKS, E_CHUNK), jnp.int32),
            pltpu.VMEM((E_CHUNK,), jnp.float32),
            pltpu.VMEM((ROWS_PER_TILE,), jnp.float32),
        ],
    )
    sc_agg1 = pl.kernel(
        _sc_agg1_body,
        out_type=jax.ShapeDtypeStruct((N_CORES, N_PAD, F_HALF), jnp.float32),
        mesh=_mesh(),
        compiler_params=pltpu.CompilerParams(use_tc_tiling_on_sc=False),
        scratch_types=[
            pltpu.VMEM_SHARED((N_PAD, F_HALF), jnp.float32),
            pltpu.VMEM((2 * N_CHUNKS, E_CHUNK), jnp.int32),
            pltpu.VMEM((2 * N_CHUNKS, E_CHUNK), jnp.int32),
            pltpu.VMEM((E_CHUNK, F_HALF), jnp.float32),
            pltpu.VMEM((E_CHUNK, F_HALF), jnp.float32),
            pltpu.SemaphoreType.DMA,
            pltpu.SemaphoreType.DMA,
            pltpu.SemaphoreType.DMA,
            pltpu.SemaphoreType.DMA,
        ],
    )
    sc_agg2 = pl.kernel(
        _sc_agg2_body,
        out_type=jax.ShapeDtypeStruct((N_CORES, N_PAD, F2), jnp.float32),
        mesh=_mesh(),
        compiler_params=pltpu.CompilerParams(use_tc_tiling_on_sc=False),
        scratch_types=[
            pltpu.VMEM_SHARED((N_PAD, F2), jnp.float32),
            pltpu.VMEM((N_CHUNKS, E_CHUNK), jnp.int32),
            pltpu.VMEM((N_CHUNKS, E_CHUNK), jnp.int32),
            pltpu.VMEM((E_CHUNK, F2), jnp.float32),
            pltpu.VMEM((E_CHUNK, F2), jnp.float32),
            pltpu.SemaphoreType.DMA,
            pltpu.SemaphoreType.DMA,
            pltpu.SemaphoreType.DMA,
            pltpu.SemaphoreType.DMA,
        ],
    )
    return sc_degree, sc_agg1, sc_agg2


def _tc1_body(x_ref, w_ref, degp_ref, p1_ref, dinv_ref):
    deg = degp_ref[:, 0:1] + degp_ref[:, 1:2] + 1.0
    dinv = lax.rsqrt(deg)
    dinv_ref[...] = dinv
    h = jnp.dot(x_ref[...], w_ref[0], preferred_element_type=jnp.float32)
    p1_ref[...] = (h * dinv)[None]


def _tc2_body(agg_ref, dinv_ref, b1_ref, w2_ref, p2_ref):
    dinv = dinv_ref[...]
    h = jnp.concatenate([agg_ref[0], agg_ref[1]], axis=1)
    r = jax.nn.relu(h * dinv + b1_ref[...])
    p2_ref[...] = jnp.dot(r, w2_ref[...],
                          preferred_element_type=jnp.float32) * dinv


def _tc3_body(agg_ref, p2_ref, dinv_ref, b2_ref, out_ref):
    tot = agg_ref[0] + agg_ref[1] - p2_ref[...]
    out_ref[...] = tot * dinv_ref[...] + b2_ref[...]


def kernel(x, edge_index, W1, b1, W2, b2):
    n, in_ch = x.shape
    ei32 = edge_index.astype(jnp.int32)
    pad_ids = 10000 + jnp.arange(E_PAD - ei32.shape[1], dtype=jnp.int32) % 240
    ei32 = jnp.concatenate(
        [ei32, jnp.stack([pad_ids, pad_ids])], axis=1)
    ei = ei32.reshape(2, N_WORKERS, N_CHUNKS, E_CHUNK)
    ei1 = ei32.reshape(2, N_TILES, 2 * N_CHUNKS, E_CHUNK)
    x_pad = jnp.pad(x, ((0, N_PAD - n), (0, 0)))
    w2_pad = jnp.pad(W2, ((0, 0), (0, F2 - W2.shape[1])))
    b2_pad = jnp.pad(b2, (0, F2 - b2.shape[0])).reshape(1, F2)
    b1_row = b1.reshape(1, -1)
    w1_split = W1.reshape(in_ch, N_CORES, F_HALF).transpose(1, 0, 2)
    _sc_degree, _sc_agg1, _sc_agg2 = _sc_kernels()

    degp = _sc_degree(ei)
    degp2 = degp.T

    p1, dinv = pl.pallas_call(
        _tc1_body,
        grid=(N_CORES,),
        in_specs=[
            pl.BlockSpec((N_PAD, in_ch), lambda c: (0, 0)),
            pl.BlockSpec((1, in_ch, F_HALF), lambda c: (c, 0, 0)),
            pl.BlockSpec((N_PAD, 2), lambda c: (0, 0)),
        ],
        out_specs=[
            pl.BlockSpec((1, N_PAD, F_HALF), lambda c: (c, 0, 0)),
            pl.BlockSpec((N_PAD, 1), lambda c: (0, 0)),
        ],
        out_shape=[
            jax.ShapeDtypeStruct((N_CORES, N_PAD, F_HALF), jnp.float32),
            jax.ShapeDtypeStruct((N_PAD, 1), jnp.float32),
        ],
    )(x_pad, w1_split, degp2)

    agg1 = _sc_agg1(p1, ei1)

    p2 = pl.pallas_call(
        _tc2_body,
        out_shape=jax.ShapeDtypeStruct((N_PAD, F2), jnp.float32),
    )(agg1, dinv, b1_row, w2_pad)

    agg2 = _sc_agg2(p2, ei)

    out = pl.pallas_call(
        _tc3_body,
        out_shape=jax.ShapeDtypeStruct((N_PAD, F2), jnp.float32),
    )(agg2, p2, dinv, b2_pad)

    return out[:n, :3]


# speedup vs baseline: 1.4436x; 1.1504x over previous
import functools

import jax
import jax.numpy as jnp
from jax import lax
from jax.experimental import pallas as pl
from jax.experimental.pallas import tpu as pltpu
from jax.experimental.pallas import tpu_sc as plsc

N_PAD = 10240
N_TILES = 16
N_CORES = 2
N_WORKERS = N_CORES * N_TILES
ROWS_PER_TILE = N_PAD // N_TILES
E_CHUNK = 112
N_CHUNKS = 90
E_PAD = N_WORKERS * N_CHUNKS * E_CHUNK
PAD_NODE = 10200
F_HALF = 64
F2 = 8

def _worker_id():
    return lax.axis_index("s") * N_CORES + lax.axis_index("c")


def _mesh():
    return plsc.VectorSubcoreMesh(
        core_axis_name="c", subcore_axis_name="s",
        num_cores=N_CORES, num_subcores=N_TILES)


def _sc_degree_body(ei, degp, acc, idx_dst, ones_v, zeros_v):
    c = lax.axis_index("c")
    s = lax.axis_index("s")
    w = _worker_id()
    for i in range(E_CHUNK // 16):
        ones_v[pl.ds(i * 16, 16)] = jnp.ones((16,), jnp.float32)
    for i in range(ROWS_PER_TILE // 16):
        zeros_v[pl.ds(i * 16, 16)] = jnp.zeros((16,), jnp.float32)
    pltpu.sync_copy(zeros_v, acc.at[pl.ds(s * ROWS_PER_TILE, ROWS_PER_TILE)])
    pltpu.sync_copy(ei.at[1, w], idx_dst)
    plsc.subcore_barrier()

    def body(i, _):
        pltpu.sync_copy(ones_v, acc.at[idx_dst.at[i]], add=True)
        return ()

    lax.fori_loop(0, N_CHUNKS, body, ())
    plsc.subcore_barrier()
    pltpu.sync_copy(acc.at[pl.ds(s * ROWS_PER_TILE, ROWS_PER_TILE)],
                    degp.at[c, pl.ds(s * ROWS_PER_TILE, ROWS_PER_TILE)])


def _pipelined_agg(table, acc, idx_src, idx_dst, bufs, sems, n_chunks):
    depth = len(bufs)
    assert n_chunks % depth == 0

    def gather(i, buf, sem):
        pltpu.async_copy(table.at[idx_src.at[i]], buf, sem)

    def wait_gather(i, buf, sem):
        pltpu.make_async_copy(table.at[idx_src.at[i]], buf, sem).wait()

    def sync_scatter(i, buf):
        pltpu.sync_copy(buf, acc.at[idx_dst.at[i]], add=True)

    gather(0, bufs[0], sems[0])
    gather(1, bufs[1], sems[1])

    def body(k, _):
        for j in range(depth):
            i = depth * k + j
            inext = i + depth - 1
            bnext = (j + depth - 1) % depth

            @pl.when(inext < n_chunks)
            def _():
                gather(inext, bufs[bnext], sems[bnext])

            wait_gather(i, bufs[j], sems[j])
            sync_scatter(i, bufs[j])
        return ()

    lax.fori_loop(0, n_chunks // depth, body, ())


def _sc_agg1_body(p1, ei1, agg, acc, idx_src, idx_dst,
                  msg0, msg1, msg2, semg0, semg1, semg2):
    c = lax.axis_index("c")
    s = lax.axis_index("s")
    rows = pl.ds(s * ROWS_PER_TILE, ROWS_PER_TILE)
    pltpu.sync_copy(p1.at[c, rows], acc.at[rows])
    pltpu.sync_copy(ei1.at[0, s], idx_src)
    pltpu.sync_copy(ei1.at[1, s], idx_dst)
    plsc.subcore_barrier()
    _pipelined_agg(p1.at[c], acc, idx_src, idx_dst,
                   [msg0, msg1, msg2], [semg0, semg1, semg2], 2 * N_CHUNKS)
    plsc.subcore_barrier()
    pltpu.sync_copy(acc.at[rows], agg.at[c, rows])


def _sc_agg2_body(p2, ei, agg, acc, idx_src, idx_dst,
                  msg0, msg1, msg2, semg0, semg1, semg2):
    c = lax.axis_index("c")
    s = lax.axis_index("s")
    w = _worker_id()
    rows = pl.ds(s * ROWS_PER_TILE, ROWS_PER_TILE)
    pltpu.sync_copy(p2.at[rows], acc.at[rows])
    pltpu.sync_copy(ei.at[0, w], idx_src)
    pltpu.sync_copy(ei.at[1, w], idx_dst)
    plsc.subcore_barrier()
    _pipelined_agg(p2, acc, idx_src, idx_dst,
                   [msg0, msg1, msg2], [semg0, semg1, semg2], N_CHUNKS)
    plsc.subcore_barrier()
    pltpu.sync_copy(acc.at[rows], agg.at[c, rows])


@functools.cache
def _sc_kernels():
    sc_degree = pl.kernel(
        _sc_degree_body,
        out_type=jax.ShapeDtypeStruct((N_CORES, N_PAD), jnp.float32),
        mesh=_mesh(),
        compiler_params=pltpu.CompilerParams(use_tc_tiling_on_sc=False),
        scratch_types=[
            pltpu.VMEM_SHARED((N_PAD,), jnp.float32),
            pltpu.VMEM((N_CHUNKS, E_CHUNK), jnp.int32),
            pltpu.VMEM((E_CHUNK,), jnp.float32),
            pltpu.VMEM((ROWS_PER_TILE,), jnp.float32),
        ],
    )
    sc_agg1 = pl.kernel(
        _sc_agg1_body,
        out_type=jax.ShapeDtypeStruct((N_CORES, N_PAD, F_HALF), jnp.float32),
        mesh=_mesh(),
        compiler_params=pltpu.CompilerParams(use_tc_tiling_on_sc=False),
        scratch_types=[
            pltpu.VMEM_SHARED((N_PAD, F_HALF), jnp.float32),
            pltpu.VMEM((2 * N_CHUNKS, E_CHUNK), jnp.int32),
            pltpu.VMEM((2 * N_CHUNKS, E_CHUNK), jnp.int32),
            pltpu.VMEM((E_CHUNK, F_HALF), jnp.float32),
            pltpu.VMEM((E_CHUNK, F_HALF), jnp.float32),
            pltpu.VMEM((E_CHUNK, F_HALF), jnp.float32),
            pltpu.SemaphoreType.DMA,
            pltpu.SemaphoreType.DMA,
            pltpu.SemaphoreType.DMA,
        ],
    )
    sc_agg2 = pl.kernel(
        _sc_agg2_body,
        out_type=jax.ShapeDtypeStruct((N_CORES, N_PAD, F2), jnp.float32),
        mesh=_mesh(),
        compiler_params=pltpu.CompilerParams(use_tc_tiling_on_sc=False),
        scratch_types=[
            pltpu.VMEM_SHARED((N_PAD, F2), jnp.float32),
            pltpu.VMEM((N_CHUNKS, E_CHUNK), jnp.int32),
            pltpu.VMEM((N_CHUNKS, E_CHUNK), jnp.int32),
            pltpu.VMEM((E_CHUNK, F2), jnp.float32),
            pltpu.VMEM((E_CHUNK, F2), jnp.float32),
            pltpu.VMEM((E_CHUNK, F2), jnp.float32),
            pltpu.SemaphoreType.DMA,
            pltpu.SemaphoreType.DMA,
            pltpu.SemaphoreType.DMA,
        ],
    )
    return sc_degree, sc_agg1, sc_agg2


def _tc1_body(x_ref, w_ref, degp_ref, p1_ref, dinv_ref):
    deg = degp_ref[:, 0:1] + degp_ref[:, 1:2] + 1.0
    dinv = lax.rsqrt(deg)
    dinv_ref[...] = dinv
    h = jnp.dot(x_ref[...], w_ref[0], preferred_element_type=jnp.float32)
    p1_ref[...] = (h * dinv)[None]


def _tc2_body(agg_ref, dinv_ref, b1_ref, w2_ref, p2_ref):
    dinv = dinv_ref[...]
    h = jnp.concatenate([agg_ref[0], agg_ref[1]], axis=1)
    r = jax.nn.relu(h * dinv + b1_ref[...])
    p2_ref[...] = jnp.dot(r, w2_ref[...],
                          preferred_element_type=jnp.float32) * dinv


def _tc3_body(agg_ref, p2_ref, dinv_ref, b2_ref, out_ref):
    tot = agg_ref[0] + agg_ref[1] - p2_ref[...]
    out_ref[...] = tot * dinv_ref[...] + b2_ref[...]


def kernel(x, edge_index, W1, b1, W2, b2):
    n, in_ch = x.shape
    ei32 = edge_index.astype(jnp.int32)
    pad_ids = 10000 + jnp.arange(E_PAD - ei32.shape[1], dtype=jnp.int32) % 240
    ei32 = jnp.concatenate(
        [ei32, jnp.stack([pad_ids, pad_ids])], axis=1)
    ei = ei32.reshape(2, N_WORKERS, N_CHUNKS, E_CHUNK)
    ei1 = ei32.reshape(2, N_TILES, 2 * N_CHUNKS, E_CHUNK)
    x_pad = jnp.pad(x, ((0, N_PAD - n), (0, 0)))
    w2_pad = jnp.pad(W2, ((0, 0), (0, F2 - W2.shape[1])))
    b2_pad = jnp.pad(b2, (0, F2 - b2.shape[0])).reshape(1, F2)
    b1_row = b1.reshape(1, -1)
    w1_split = W1.reshape(in_ch, N_CORES, F_HALF).transpose(1, 0, 2)
    _sc_degree, _sc_agg1, _sc_agg2 = _sc_kernels()

    degp = _sc_degree(ei)
    degp2 = degp.T

    p1, dinv = pl.pallas_call(
        _tc1_body,
        grid=(N_CORES,),
        in_specs=[
            pl.BlockSpec((N_PAD, in_ch), lambda c: (0, 0)),
            pl.BlockSpec((1, in_ch, F_HALF), lambda c: (c, 0, 0)),
            pl.BlockSpec((N_PAD, 2), lambda c: (0, 0)),
        ],
        out_specs=[
            pl.BlockSpec((1, N_PAD, F_HALF), lambda c: (c, 0, 0)),
            pl.BlockSpec((N_PAD, 1), lambda c: (0, 0)),
        ],
        out_shape=[
            jax.ShapeDtypeStruct((N_CORES, N_PAD, F_HALF), jnp.float32),
            jax.ShapeDtypeStruct((N_PAD, 1), jnp.float32),
        ],
    )(x_pad, w1_split, degp2)

    agg1 = _sc_agg1(p1, ei1)

    p2 = pl.pallas_call(
        _tc2_body,
        out_shape=jax.ShapeDtypeStruct((N_PAD, F2), jnp.float32),
    )(agg1, dinv, b1_row, w2_pad)

    agg2 = _sc_agg2(p2, ei)

    out = pl.pallas_call(
        _tc3_body,
        out_shape=jax.ShapeDtypeStruct((N_PAD, F2), jnp.float32),
    )(agg2, p2, dinv, b2_pad)

    return out[:n, :3]
